# baseline (speedup 1.0000x reference)
import jax
import jax.numpy as jnp
from jax import lax
from jax.experimental import pallas as pl
from jax.experimental.pallas import tpu as pltpu
from jax.experimental.pallas import tpu_sc as plsc

N = 10000
E = 320000
D_FEAT = 128
D_EDGE = 16
D_AUG = 32

NC = 2
NS = 16
NW = NC * NS
EPW = E // NW
CHUNK = 40
NCHUNK = EPW // CHUNK
SLAB = 5
SPAN = SLAB * CHUNK
NSLAB = NCHUNK // SLAB
NBODY = NCHUNK // (2 * SLAB)
FLUSH = 624
TAIL0 = NS * FLUSH
TAILR = N - TAIL0


def _zero_acc(sid, stage, acc):
    row0 = sid * FLUSH
    for r in range(FLUSH // CHUNK):
        pltpu.sync_copy(stage, acc.at[pl.ds(row0 + r * CHUNK, CHUNK)])
    zrem = FLUSH - (FLUSH // CHUNK) * CHUNK
    pltpu.sync_copy(stage.at[pl.ds(0, zrem)],
                    acc.at[pl.ds(row0 + FLUSH - zrem, zrem)])

    @pl.when(sid == NS - 1)
    def _zero_tail():
        pltpu.sync_copy(stage.at[pl.ds(0, TAILR)], acc.at[pl.ds(TAIL0, TAILR)])


def _flush_acc(cid, sid, acc, out):
    row0 = sid * FLUSH
    pltpu.sync_copy(acc.at[pl.ds(row0, FLUSH)],
                    out.at[cid, pl.ds(row0, FLUSH)])

    @pl.when(sid == NS - 1)
    def _flush_tail():
        pltpu.sync_copy(acc.at[pl.ds(TAIL0, TAILR)],
                        out.at[cid, pl.ds(TAIL0, TAILR)])


def _pipeline(body_step, issue_in_first):
    issue_in_first()

    def body(i, _):
        for js in range(2):
            for j in range(SLAB):
                body_step(i, js, j)
        return 0

    lax.fori_loop(0, NBODY, body, 0)


def _sc_x_body(x_hbm, ei_hbm, accx_out,
               slab0, slab1, rows0, rows1, acc_x,
               isem0, isem1, osem0, osem1, ssem0, ssem1):
    cid = lax.axis_index("c")
    sid = lax.axis_index("s")
    wid = cid * NS + sid

    slab = (slab0, slab1)
    rows = (rows0, rows1)
    isem = (isem0, isem1)
    osem = (osem0, osem1)
    ssem = (ssem0, ssem1)

    def zero_row(i, _):
        for j in range(D_FEAT // 16):
            rows0[i, pl.ds(j * 16, 16)] = jnp.zeros((16,), jnp.float32)
        return 0

    lax.fori_loop(0, CHUNK, zero_row, 0)
    _zero_acc(sid, rows0, acc_x)
    plsc.subcore_barrier()

    def issue_in(b, S, j):
        pltpu.async_copy(x_hbm.at[S.at[1, pl.ds(j * CHUNK, CHUNK)]],
                         rows[b], isem[b])

    def wait_in(b, S, j):
        pltpu.make_async_copy(x_hbm.at[S.at[1, pl.ds(j * CHUNK, CHUNK)]],
                              rows[b], isem[b]).wait()

    def issue_out(b, S, j):
        pltpu.async_copy(rows[b], acc_x.at[S.at[0, pl.ds(j * CHUNK, CHUNK)]],
                         osem[b], add=True)

    def wait_out(b):
        pltpu.make_async_copy(rows[b], acc_x.at[slab0.at[0, pl.ds(0, CHUNK)]],
                              osem[b]).wait()

    def issue_slab(s, sb):
        base = wid * EPW + s * SPAN
        pltpu.async_copy(ei_hbm.at[0, pl.ds(base, SPAN)],
                         slab[sb].at[0], ssem[sb])
        pltpu.async_copy(ei_hbm.at[1, pl.ds(base, SPAN)],
                         slab[sb].at[1], ssem[sb])

    def wait_slab(s, sb):
        base = wid * EPW + s * SPAN
        pltpu.make_async_copy(ei_hbm.at[0, pl.ds(base, SPAN)],
                              slab[sb].at[0], ssem[sb]).wait()
        pltpu.make_async_copy(ei_hbm.at[1, pl.ds(base, SPAN)],
                              slab[sb].at[1], ssem[sb]).wait()

    issue_slab(0, 0)
    wait_slab(0, 0)
    issue_in(0, slab0, 0)

    def step(i, js, j):
        S = slab[js]
        b = (js + j) % 2
        wait_in(b, S, j)
        issue_out(b, S, j)
        if js == 0 and j == 0:
            @pl.when(i > 0)
            def _():
                wait_out(1 - b)
        else:
            wait_out(1 - b)
        if js == 0 and j == 1:
            issue_slab(2 * i + 1, 1)
        if js == 1 and j == 1:
            @pl.when(i < NBODY - 1)
            def _():
                issue_slab(2 * i + 2, 0)
        if j == SLAB - 1:
            if js == 0:
                wait_slab(2 * i + 1, 1)
                issue_in(1 - b, slab1, 0)
            else:
                @pl.when(i < NBODY - 1)
                def _():
                    wait_slab(2 * i + 2, 0)
                    issue_in(1 - b, slab0, 0)
        else:
            issue_in(1 - b, S, j + 1)

    def body(i, _):
        for js in range(2):
            for j in range(SLAB):
                step(i, js, j)
        return 0

    lax.fori_loop(0, NBODY, body, 0)
    wait_out(1)
    plsc.subcore_barrier()
    _flush_acc(cid, sid, acc_x, accx_out)


_sc_segsum_x = pl.kernel(
    _sc_x_body,
    out_type=jax.ShapeDtypeStruct((NC, N, D_FEAT), jnp.float32),
    mesh=plsc.VectorSubcoreMesh(core_axis_name="c", subcore_axis_name="s"),
    scratch_types=[
        pltpu.VMEM((2, SPAN), jnp.int32),
        pltpu.VMEM((2, SPAN), jnp.int32),
        pltpu.VMEM((CHUNK, D_FEAT), jnp.float32),
        pltpu.VMEM((CHUNK, D_FEAT), jnp.float32),
        pltpu.VMEM_SHARED((N, D_FEAT), jnp.float32),
        pltpu.SemaphoreType.DMA,
        pltpu.SemaphoreType.DMA,
        pltpu.SemaphoreType.DMA,
        pltpu.SemaphoreType.DMA,
        pltpu.SemaphoreType.DMA,
        pltpu.SemaphoreType.DMA,
    ],
    compiler_params=pltpu.CompilerParams(use_tc_tiling_on_sc=False),
)


def _sc_a_body(ei_hbm, ea_hbm, acca_out,
               slab0, slab1, abuf0, abuf1, acc_a,
               isem0, isem1, osem0, osem1, ssem0, ssem1):
    cid = lax.axis_index("c")
    sid = lax.axis_index("s")
    wid = cid * NS + sid

    slab = (slab0, slab1)
    abuf = (abuf0, abuf1)
    isem = (isem0, isem1)
    osem = (osem0, osem1)
    ssem = (ssem0, ssem1)

    def zero_row(i, _):
        for j in range(D_AUG // 16):
            abuf0[i, pl.ds(j * 16, 16)] = jnp.zeros((16,), jnp.float32)
        return 0

    lax.fori_loop(0, CHUNK, zero_row, 0)
    _zero_acc(sid, abuf0, acc_a)

    onecol = jnp.where(lax.iota(jnp.int32, 16) == 0,
                       jnp.float32(1.0), jnp.float32(0.0))

    def init_abuf(i, _):
        abuf0[i, pl.ds(16, 16)] = onecol
        abuf1[i, pl.ds(16, 16)] = onecol
        return 0

    lax.fori_loop(0, CHUNK, init_abuf, 0)
    plsc.subcore_barrier()

    def issue_in(c, b):
        pltpu.async_copy(ea_hbm.at[pl.ds(wid * EPW + c * CHUNK, CHUNK)],
                         abuf[b].at[:, pl.ds(0, D_EDGE)], isem[b])

    def wait_in(c, b):
        pltpu.make_async_copy(ea_hbm.at[pl.ds(wid * EPW + c * CHUNK, CHUNK)],
                              abuf[b].at[:, pl.ds(0, D_EDGE)], isem[b]).wait()

    def issue_out(b, S, j):
        pltpu.async_copy(abuf[b], acc_a.at[S.at[0, pl.ds(j * CHUNK, CHUNK)]],
                         osem[b], add=True)

    def wait_out(b):
        pltpu.make_async_copy(abuf[b], acc_a.at[slab0.at[0, pl.ds(0, CHUNK)]],
                              osem[b]).wait()

    def issue_slab(s, sb):
        base = wid * EPW + s * SPAN
        pltpu.async_copy(ei_hbm.at[0, pl.ds(base, SPAN)],
                         slab[sb].at[0], ssem[sb])

    def wait_slab(s, sb):
        base = wid * EPW + s * SPAN
        pltpu.make_async_copy(ei_hbm.at[0, pl.ds(base, SPAN)],
                              slab[sb].at[0], ssem[sb]).wait()

    issue_slab(0, 0)
    wait_slab(0, 0)
    issue_in(0, 0)

    def step(i, js, j):
        S = slab[js]
        c = 10 * i + 5 * js + j
        b = (js + j) % 2
        wait_in(c, b)
        issue_out(b, S, j)
        if js == 0 and j == 0:
            @pl.when(i > 0)
            def _():
                wait_out(1 - b)
        else:
            wait_out(1 - b)
        if js == 0 and j == 1:
            issue_slab(2 * i + 1, 1)
        if js == 1 and j == 1:
            @pl.when(i < NBODY - 1)
            def _():
                issue_slab(2 * i + 2, 0)
        if j == SLAB - 1:
            if js == 0:
                wait_slab(2 * i + 1, 1)
                issue_in(c + 1, 1 - b)
            else:
                @pl.when(i < NBODY - 1)
                def _():
                    wait_slab(2 * i + 2, 0)
                    issue_in(c + 1, 1 - b)
        else:
            issue_in(c + 1, 1 - b)

    def body(i, _):
        for js in range(2):
            for j in range(SLAB):
                step(i, js, j)
        return 0

    lax.fori_loop(0, NBODY, body, 0)
    wait_out(1)
    plsc.subcore_barrier()
    _flush_acc(cid, sid, acc_a, acca_out)


_sc_segsum_a = pl.kernel(
    _sc_a_body,
    out_type=jax.ShapeDtypeStruct((NC, N, D_AUG), jnp.float32),
    mesh=plsc.VectorSubcoreMesh(core_axis_name="c", subcore_axis_name="s"),
    scratch_types=[
        pltpu.VMEM((1, SPAN), jnp.int32),
        pltpu.VMEM((1, SPAN), jnp.int32),
        pltpu.VMEM((CHUNK, D_AUG), jnp.float32),
        pltpu.VMEM((CHUNK, D_AUG), jnp.float32),
        pltpu.VMEM_SHARED((N, D_AUG), jnp.float32),
        pltpu.SemaphoreType.DMA,
        pltpu.SemaphoreType.DMA,
        pltpu.SemaphoreType.DMA,
        pltpu.SemaphoreType.DMA,
        pltpu.SemaphoreType.DMA,
        pltpu.SemaphoreType.DMA,
    ],
    compiler_params=pltpu.CompilerParams(use_tc_tiling_on_sc=False),
)


def _tc_body(x_ref, ax_ref, aa_ref, w1_ref, w2_ref, w5_ref, b1_ref, o_ref):
    acc = jnp.dot(x_ref[...], w1_ref[...], preferred_element_type=jnp.float32)
    acc += jnp.dot(ax_ref[0] + ax_ref[1], w2_ref[...],
                   preferred_element_type=jnp.float32)
    acc += jnp.dot(aa_ref[0] + aa_ref[1], w5_ref[...],
                   preferred_element_type=jnp.float32)
    o_ref[...] = acc + b1_ref[...]


ROW_BLK = 1000


def _tc_combine(x, accx, acca, W1, W2, W5aug, b1):
    return pl.pallas_call(
        _tc_body,
        out_shape=jax.ShapeDtypeStruct((N, D_FEAT), jnp.float32),
        grid=(N // ROW_BLK,),
        in_specs=[
            pl.BlockSpec((ROW_BLK, D_FEAT), lambda i: (i, 0)),
            pl.BlockSpec((NC, ROW_BLK, D_FEAT), lambda i: (0, i, 0)),
            pl.BlockSpec((NC, ROW_BLK, D_AUG), lambda i: (0, i, 0)),
            pl.BlockSpec((D_FEAT, D_FEAT), lambda i: (0, 0)),
            pl.BlockSpec((D_FEAT, D_FEAT), lambda i: (0, 0)),
            pl.BlockSpec((D_AUG, D_FEAT), lambda i: (0, 0)),
            pl.BlockSpec((1, D_FEAT), lambda i: (0, 0)),
        ],
        out_specs=pl.BlockSpec((ROW_BLK, D_FEAT), lambda i: (i, 0)),
    )(x, accx, acca, W1, W2, W5aug, b1)


def kernel(x, edge_index, edge_attr, W1, b1, W2, b2, W3, b3, W4, b4, W5, b5):
    accx = _sc_segsum_x(x, edge_index)
    acca = _sc_segsum_a(edge_index, edge_attr)
    W5aug = jnp.zeros((D_AUG, D_FEAT), jnp.float32).at[0:16].set(W5).at[16].set(b2 + b5)
    return _tc_combine(x, accx, acca, W1, W2, W5aug, b1.reshape(1, D_FEAT))


# speedup vs baseline: 1.6946x; 1.6946x over previous
import jax
import jax.numpy as jnp
from jax import lax
from jax.experimental import pallas as pl
from jax.experimental.pallas import tpu as pltpu
from jax.experimental.pallas import tpu_sc as plsc

N = 10000
E = 320000
D_FEAT = 128
D_EDGE = 16
D_AUG = 32

NC = 2
NS = 16
NW = NC * NS
EPW = E // NW
CHUNK = 80
NCHUNK = EPW // CHUNK
SLAB = 5
SPAN = SLAB * CHUNK
NSLAB = NCHUNK // SLAB
NBODY = 12
FLUSH = 624
TAIL0 = NS * FLUSH
TAILR = N - TAIL0


def _sc_body(x_hbm, ei_hbm, ea_hbm, accx_out, acca_out,
             slab0, slab1, rows0, rows1, abuf0, abuf1,
             acc_x, acc_a, isem0, isem1, osem0, osem1, ssem0, ssem1):
    cid = lax.axis_index("c")
    sid = lax.axis_index("s")
    wid = cid * NS + sid

    slab = (slab0, slab1)
    rows = (rows0, rows1)
    abuf = (abuf0, abuf1)
    isem = (isem0, isem1)
    osem = (osem0, osem1)
    ssem = (ssem0, ssem1)

    def zero_row(i, _):
        for j in range(D_FEAT // 16):
            rows0[i, pl.ds(j * 16, 16)] = jnp.zeros((16,), jnp.float32)
        for j in range(D_AUG // 16):
            abuf0[i, pl.ds(j * 16, 16)] = jnp.zeros((16,), jnp.float32)
        return 0

    lax.fori_loop(0, CHUNK, zero_row, 0)
    row0 = sid * FLUSH
    for r in range(FLUSH // CHUNK):
        pltpu.sync_copy(rows0, acc_x.at[pl.ds(row0 + r * CHUNK, CHUNK)])
        pltpu.sync_copy(abuf0, acc_a.at[pl.ds(row0 + r * CHUNK, CHUNK)])
    zrem = FLUSH - (FLUSH // CHUNK) * CHUNK
    pltpu.sync_copy(rows0.at[pl.ds(0, zrem)],
                    acc_x.at[pl.ds(row0 + FLUSH - zrem, zrem)])
    pltpu.sync_copy(abuf0.at[pl.ds(0, zrem)],
                    acc_a.at[pl.ds(row0 + FLUSH - zrem, zrem)])

    @pl.when(sid == NS - 1)
    def _zero_tail():
        pltpu.sync_copy(rows0.at[pl.ds(0, TAILR)],
                        acc_x.at[pl.ds(TAIL0, TAILR)])
        pltpu.sync_copy(abuf0.at[pl.ds(0, TAILR)],
                        acc_a.at[pl.ds(TAIL0, TAILR)])

    onecol = jnp.where(lax.iota(jnp.int32, 16) == 0,
                       jnp.float32(1.0), jnp.float32(0.0))

    def init_abuf(i, _):
        abuf0[i, pl.ds(16, 16)] = onecol
        abuf1[i, pl.ds(16, 16)] = onecol
        return 0

    lax.fori_loop(0, CHUNK, init_abuf, 0)

    plsc.subcore_barrier()

    def issue_in(c, b, S, j):
        pltpu.async_copy(x_hbm.at[S.at[1, pl.ds(j * CHUNK, CHUNK)]],
                         rows[b], isem[b])
        pltpu.async_copy(ea_hbm.at[pl.ds(wid * EPW + c * CHUNK, CHUNK)],
                         abuf[b].at[:, pl.ds(0, D_EDGE)], isem[b])

    def wait_in(c, b, S, j):
        pltpu.make_async_copy(x_hbm.at[S.at[1, pl.ds(j * CHUNK, CHUNK)]],
                              rows[b], isem[b]).wait()
        pltpu.make_async_copy(ea_hbm.at[pl.ds(wid * EPW + c * CHUNK, CHUNK)],
                              abuf[b].at[:, pl.ds(0, D_EDGE)], isem[b]).wait()

    def issue_out(b, S, j):
        pltpu.async_copy(rows[b], acc_x.at[S.at[0, pl.ds(j * CHUNK, CHUNK)]],
                         osem[b], add=True)
        pltpu.async_copy(abuf[b], acc_a.at[S.at[0, pl.ds(j * CHUNK, CHUNK)]],
                         osem[b], add=True)

    def wait_out(b):
        pltpu.make_async_copy(rows[b], acc_x.at[slab0.at[0, pl.ds(0, CHUNK)]],
                              osem[b]).wait()
        pltpu.make_async_copy(abuf[b], acc_a.at[slab0.at[0, pl.ds(0, CHUNK)]],
                              osem[b]).wait()

    def issue_slab(s, sb):
        base = wid * EPW + s * SPAN
        pltpu.async_copy(ei_hbm.at[0, pl.ds(base, SPAN)],
                         slab[sb].at[0], ssem[sb])
        pltpu.async_copy(ei_hbm.at[1, pl.ds(base, SPAN)],
                         slab[sb].at[1], ssem[sb])

    def wait_slab(s, sb):
        base = wid * EPW + s * SPAN
        pltpu.make_async_copy(ei_hbm.at[0, pl.ds(base, SPAN)],
                              slab[sb].at[0], ssem[sb]).wait()
        pltpu.make_async_copy(ei_hbm.at[1, pl.ds(base, SPAN)],
                              slab[sb].at[1], ssem[sb]).wait()

    issue_slab(0, 0)
    wait_slab(0, 0)
    issue_in(0, 0, slab0, 0)

    def step(i, js, j):
        S = slab[js]
        c = 10 * i + 5 * js + j
        b = (js + j) % 2
        wait_in(c, b, S, j)
        issue_out(b, S, j)
        if js == 0 and j == 0:
            @pl.when(i > 0)
            def _():
                wait_out(1 - b)
        else:
            wait_out(1 - b)
        if js == 0 and j == 1:
            issue_slab(2 * i + 1, 1)
        if js == 1 and j == 1:
            issue_slab(2 * i + 2, 0)
        if j == SLAB - 1:
            if js == 0:
                wait_slab(2 * i + 1, 1)
                issue_in(c + 1, 1 - b, slab1, 0)
            else:
                wait_slab(2 * i + 2, 0)
                issue_in(c + 1, 1 - b, slab0, 0)
        else:
            issue_in(c + 1, 1 - b, S, j + 1)

    def body(i, _):
        for js in range(2):
            for j in range(SLAB):
                step(i, js, j)
        return 0

    lax.fori_loop(0, NBODY, body, 0)

    for j in range(SLAB):
        c = 10 * NBODY + j
        b = j % 2
        wait_in(c, b, slab0, j)
        issue_out(b, slab0, j)
        wait_out(1 - b)
        if j < SLAB - 1:
            issue_in(c + 1, 1 - b, slab0, j + 1)
    wait_out(0)

    plsc.subcore_barrier()

    pltpu.sync_copy(acc_x.at[pl.ds(row0, FLUSH)],
                    accx_out.at[cid, pl.ds(row0, FLUSH)])
    pltpu.sync_copy(acc_a.at[pl.ds(row0, FLUSH)],
                    acca_out.at[cid, pl.ds(row0, FLUSH)])

    @pl.when(sid == NS - 1)
    def _flush_tail():
        pltpu.sync_copy(acc_x.at[pl.ds(TAIL0, TAILR)],
                        accx_out.at[cid, pl.ds(TAIL0, TAILR)])
        pltpu.sync_copy(acc_a.at[pl.ds(TAIL0, TAILR)],
                        acca_out.at[cid, pl.ds(TAIL0, TAILR)])


_sc_segsum = pl.kernel(
    _sc_body,
    out_type=(
        jax.ShapeDtypeStruct((NC, N, D_FEAT), jnp.float32),
        jax.ShapeDtypeStruct((NC, N, D_AUG), jnp.float32),
    ),
    mesh=plsc.VectorSubcoreMesh(core_axis_name="c", subcore_axis_name="s"),
    scratch_types=[
        pltpu.VMEM((2, SPAN), jnp.int32),
        pltpu.VMEM((2, SPAN), jnp.int32),
        pltpu.VMEM((CHUNK, D_FEAT), jnp.float32),
        pltpu.VMEM((CHUNK, D_FEAT), jnp.float32),
        pltpu.VMEM((CHUNK, D_AUG), jnp.float32),
        pltpu.VMEM((CHUNK, D_AUG), jnp.float32),
        pltpu.VMEM_SHARED((N, D_FEAT), jnp.float32),
        pltpu.VMEM_SHARED((N, D_AUG), jnp.float32),
        pltpu.SemaphoreType.DMA,
        pltpu.SemaphoreType.DMA,
        pltpu.SemaphoreType.DMA,
        pltpu.SemaphoreType.DMA,
        pltpu.SemaphoreType.DMA,
        pltpu.SemaphoreType.DMA,
    ],
    compiler_params=pltpu.CompilerParams(use_tc_tiling_on_sc=False),
)


def _tc_body(x_ref, ax_ref, aa_ref, w1_ref, w2_ref, w5_ref, b1_ref, o_ref):
    acc = jnp.dot(x_ref[...], w1_ref[...], preferred_element_type=jnp.float32)
    acc += jnp.dot(ax_ref[0] + ax_ref[1], w2_ref[...],
                   preferred_element_type=jnp.float32)
    acc += jnp.dot(aa_ref[0] + aa_ref[1], w5_ref[...],
                   preferred_element_type=jnp.float32)
    o_ref[...] = acc + b1_ref[...]


ROW_BLK = 1000


def _tc_combine(x, accx, acca, W1, W2, W5aug, b1):
    return pl.pallas_call(
        _tc_body,
        out_shape=jax.ShapeDtypeStruct((N, D_FEAT), jnp.float32),
        grid=(N // ROW_BLK,),
        in_specs=[
            pl.BlockSpec((ROW_BLK, D_FEAT), lambda i: (i, 0)),
            pl.BlockSpec((NC, ROW_BLK, D_FEAT), lambda i: (0, i, 0)),
            pl.BlockSpec((NC, ROW_BLK, D_AUG), lambda i: (0, i, 0)),
            pl.BlockSpec((D_FEAT, D_FEAT), lambda i: (0, 0)),
            pl.BlockSpec((D_FEAT, D_FEAT), lambda i: (0, 0)),
            pl.BlockSpec((D_AUG, D_FEAT), lambda i: (0, 0)),
            pl.BlockSpec((1, D_FEAT), lambda i: (0, 0)),
        ],
        out_specs=pl.BlockSpec((ROW_BLK, D_FEAT), lambda i: (i, 0)),
    )(x, accx, acca, W1, W2, W5aug, b1)


def kernel(x, edge_index, edge_attr, W1, b1, W2, b2, W3, b3, W4, b4, W5, b5):
    accx, acca = _sc_segsum(x, edge_index, edge_attr)
    W5aug = jnp.zeros((D_AUG, D_FEAT), jnp.float32).at[0:16].set(W5).at[16].set(b2 + b5)
    return _tc_combine(x, accx, acca, W1, W2, W5aug, b1.reshape(1, D_FEAT))
